# baseline (speedup 1.0000x reference)
import functools

import jax
import jax.numpy as jnp
from jax import lax
from jax.experimental import pallas as pl
from jax.experimental.pallas import tpu as pltpu
from jax.experimental.pallas import tpu_sc as plsc

N = 10000
E = 320000
D_IN = 128
H = 64
B = 8
L = 3
D_OUT = 128

N_PAD = 10240
E_PAD = 327680
TILE_N = 2048
TILE_E = 2048
N_BLKS = N_PAD // TILE_N
E_BLKS = E_PAD // TILE_E

IDX_ROW = 128
ROWS_PER_TILE = (E_PAD // IDX_ROW) // 32
R_CHUNK = 2
N_CHUNKS = ROWS_PER_TILE // R_CHUNK
SLICE_N = N_PAD // 16
BUF_ROWS = 40

_f32 = jnp.float32


def _dot(a, b):
    return jnp.dot(a, b, preferred_element_type=_f32)


def _full(shape):
    return pl.BlockSpec(shape, lambda i: tuple(0 for _ in shape))


def _tables_body(cond_ref, ceW1, ceb1, ceW2, ceb2, aWv, abv, aWo, abo,
                 W1d_all, peb1_all, Wnt_all, pnb1_all, ug_out, attc_out):
    u = _dot(jnp.maximum(_dot(cond_ref[...], ceW1[...]) + ceb1[...], 0.0),
             ceW2[...]) + ceb2[...]
    att_g = _dot(_dot(u, aWv[...]) + abv[...], aWo[...]) + abo[...]
    for l in range(L):
        ug_out[l] = _dot(u, W1d_all[l]) + peb1_all[l]
        attc_out[l] = _dot(att_g, Wnt_all[l]) + pnb1_all[l]


def _encode_body(x_ref, W1, b1, W2, b2, h_ref):
    t = jnp.maximum(_dot(x_ref[...], W1[...]) + b1[...], 0.0)
    h_ref[...] = _dot(t, W2[...]) + b2[...]


def _onehot(b3_ref):
    b = b3_ref[0, 0, :]
    io = lax.broadcasted_iota(jnp.int32, (1, B), 1)
    return (b[:, None] == io).astype(_f32)


def _prep_body(h_ref, b3_ref, ug_ref, W1a, W1b, tab_ref):
    h = h_ref[...]
    oh = _onehot(b3_ref)
    ha2 = _dot(h, W1a[...]) + _dot(oh, ug_ref[...])
    hb = _dot(h, W1b[...])
    tab_ref[...] = jnp.concatenate([ha2, hb], axis=1)


def _edge0_body(g1_ref, g2_ref, attr_ref, eeW1, eeb1, eeW2, eeb2,
                W1c, W2, b2, out_ref):
    e0 = _dot(jnp.maximum(_dot(attr_ref[...], eeW1[...]) + eeb1[...], 0.0),
              eeW2[...]) + eeb2[...]
    pre = g1_ref[:, :H] + g2_ref[:, H:] + _dot(e0, W1c[...])
    ea = _dot(jnp.maximum(pre, 0.0), W2[...]) + b2[...]
    out_ref[...] = jnp.concatenate([ea, jnp.zeros_like(ea)], axis=1)


def _edge_body(g1_ref, g2_ref, ea_ref, W1c, W2, b2, out_ref):
    pre = g1_ref[:, :H] + g2_ref[:, H:] + _dot(ea_ref[:, :H], W1c[...])
    ea = _dot(jnp.maximum(pre, 0.0), W2[...]) + b2[...]
    out_ref[...] = jnp.concatenate([ea, jnp.zeros_like(ea)], axis=1)


def _node_body(h_ref, aggp_ref, b3_ref, attc_ref, Wnh, Wna, W2, b2, out_ref):
    h = h_ref[...]
    agg = aggp_ref[0, :, :H] + aggp_ref[1, :, :H]
    oh = _onehot(b3_ref)
    npre = _dot(h, Wnh[...]) + _dot(agg, Wna[...]) + _dot(oh, attc_ref[...])
    out_ref[...] = _dot(jnp.maximum(npre, 0.0), W2[...]) + b2[...] + h


def _row_spec(w=H):
    return pl.BlockSpec((TILE_N, w), lambda i: (i, 0))


def _b3_spec():
    return pl.BlockSpec((1, 1, TILE_N), lambda i: (i, 0, 0))


_MESH = plsc.VectorSubcoreMesh(core_axis_name="c", subcore_axis_name="s")


@functools.partial(
    pl.kernel,
    out_type=(jax.ShapeDtypeStruct((E_PAD, 2 * H), _f32),
              jax.ShapeDtypeStruct((E_PAD, 2 * H), _f32)),
    mesh=_MESH,
    scratch_types=[
        pltpu.VMEM((ROWS_PER_TILE, IDX_ROW), jnp.int32),
        pltpu.VMEM((ROWS_PER_TILE, IDX_ROW), jnp.int32),
        pltpu.VMEM((IDX_ROW, 2 * H), _f32),
        pltpu.VMEM((IDX_ROW, 2 * H), _f32),
        pltpu.VMEM((IDX_ROW, 2 * H), _f32),
        pltpu.VMEM((IDX_ROW, 2 * H), _f32),
        pltpu.VMEM((IDX_ROW, 2 * H), _f32),
        pltpu.VMEM((IDX_ROW, 2 * H), _f32),
        pltpu.SemaphoreType.DMA, pltpu.SemaphoreType.DMA,
        pltpu.SemaphoreType.DMA, pltpu.SemaphoreType.DMA,
        pltpu.SemaphoreType.DMA, pltpu.SemaphoreType.DMA,
        pltpu.SemaphoreType.DMA, pltpu.SemaphoreType.DMA,
        pltpu.SemaphoreType.DMA, pltpu.SemaphoreType.DMA,
        pltpu.SemaphoreType.DMA, pltpu.SemaphoreType.DMA,
    ],
)
def _sc_gather2(tab_hbm, ia_hbm, ib_hbm, ga_hbm, gb_hbm,
                ia_v, ib_v, b0, b1, b2, b3, b4, b5,
                g0, g1, g2, g3, g4, g5, w0, w1, w2, w3, w4, w5):
    wid = lax.axis_index("s") * 2 + lax.axis_index("c")
    base_row = wid * ROWS_PER_TILE
    bufs = (b0, b1, b2, b3, b4, b5)
    gsems = (g0, g1, g2, g3, g4, g5)
    wsems = (w0, w1, w2, w3, w4, w5)

    pltpu.sync_copy(ia_hbm.at[pl.ds(base_row, ROWS_PER_TILE)], ia_v)
    pltpu.sync_copy(ib_hbm.at[pl.ds(base_row, ROWS_PER_TILE)], ib_v)

    def _idx(k):
        return ia_v if k % 2 == 0 else ib_v

    def _out(k):
        return ga_hbm if k % 2 == 0 else gb_hbm

    def _fire_g(k, r):
        pltpu.async_copy(tab_hbm.at[_idx(k).at[r]], bufs[k], gsems[k])

    def _wait_g(k):
        pltpu.make_async_copy(tab_hbm.at[_idx(k).at[0]], bufs[k],
                              gsems[k]).wait()

    def _fire_w(k, r):
        dst = _out(k).at[pl.ds((base_row + r) * IDX_ROW, IDX_ROW)]
        pltpu.async_copy(bufs[k], dst, wsems[k])

    def _wait_w(k):
        dst = _out(k).at[pl.ds(0, IDX_ROW)]
        pltpu.make_async_copy(bufs[k], dst, wsems[k]).wait()

    def body(i, _):
        for k in range(6):
            r = 3 * i + k // 2

            @pl.when(i > 0)
            def _():
                _wait_w(k)

            _fire_g(k, r)
        for k in range(6):
            _wait_g(k)
            _fire_w(k, 3 * i + k // 2)
        return 0

    n_full = ROWS_PER_TILE // 3
    lax.fori_loop(0, n_full, body, 0)
    tail0 = 3 * n_full
    for k in range(4):
        _wait_w(k)
        _fire_g(k, tail0 + k // 2)
    for k in range(4):
        _wait_g(k)
        _fire_w(k, tail0 + k // 2)
    for k in range(6):
        _wait_w(k)


@functools.partial(
    pl.kernel,
    out_type=jax.ShapeDtypeStruct((2 * N_PAD, 2 * H), _f32),
    mesh=_MESH,
    scratch_types=[
        pltpu.VMEM((ROWS_PER_TILE, IDX_ROW), jnp.int32),
        pltpu.VMEM((IDX_ROW, 2 * H), _f32),
        pltpu.VMEM((IDX_ROW, 2 * H), _f32),
        pltpu.VMEM((BUF_ROWS, 2 * H), _f32),
        pltpu.VMEM_SHARED((N_PAD, 2 * H), _f32),
        pltpu.SemaphoreType.DMA, pltpu.SemaphoreType.DMA,
        pltpu.SemaphoreType.DMA, pltpu.SemaphoreType.DMA,
    ],
)
def _sc_scatter_add(ea_hbm, idx_hbm, zero_hbm, out_hbm,
                    idx_v, d0, d1, buf_v, acc_sh, l0, l1, a0, a1):
    cid = lax.axis_index("c")
    sid = lax.axis_index("s")
    wid = sid * 2 + cid
    base_row = wid * ROWS_PER_TILE
    bufs = (d0, d1)
    lsems = (l0, l1)
    asems = (a0, a1)

    pltpu.sync_copy(idx_hbm.at[pl.ds(base_row, ROWS_PER_TILE)], idx_v)
    pltpu.sync_copy(zero_hbm, buf_v)

    def zinit(p, _):
        pltpu.sync_copy(
            buf_v, acc_sh.at[pl.ds(sid * SLICE_N + p * BUF_ROWS, BUF_ROWS)])
        return 0

    lax.fori_loop(0, SLICE_N // BUF_ROWS, zinit, 0)
    plsc.subcore_barrier()

    def _fire_l(p, r):
        src = ea_hbm.at[pl.ds((base_row + r) * IDX_ROW, IDX_ROW)]
        pltpu.async_copy(src, bufs[p], lsems[p])

    def _wait_l(p):
        src = ea_hbm.at[pl.ds(0, IDX_ROW)]
        pltpu.make_async_copy(src, bufs[p], lsems[p]).wait()

    def _fire_a(p, r):
        pltpu.async_copy(bufs[p], acc_sh.at[idx_v.at[r]], asems[p], add=True)

    def _wait_a(p):
        pltpu.make_async_copy(bufs[p], acc_sh.at[idx_v.at[0]],
                              asems[p]).wait()

    def body(i, _):
        for p in range(2):
            r = 2 * i + p

            @pl.when(i > 0)
            def _():
                _wait_a(p)

            _fire_l(p, r)
        for p in range(2):
            _wait_l(p)
            _fire_a(p, 2 * i + p)
        return 0

    lax.fori_loop(0, ROWS_PER_TILE // 2, body, 0)
    for p in range(2):
        _wait_a(p)
    plsc.subcore_barrier()

    def writeout(p, _):
        r = sid * SLICE_N + p * BUF_ROWS
        pltpu.sync_copy(acc_sh.at[pl.ds(r, BUF_ROWS)], buf_v)
        pltpu.sync_copy(buf_v, out_hbm.at[pl.ds(cid * N_PAD + r, BUF_ROWS)])
        return 0

    lax.fori_loop(0, SLICE_N // BUF_ROWS, writeout, 0)


def kernel(x, edge_index, edge_attr, conditions, batch,
           ne_W1, ne_b1, ne_W2, ne_b2, ee_W1, ee_b1, ee_W2, ee_b2,
           ce_W1, ce_b1, ce_W2, ce_b2,
           a_Wq, a_bq, a_Wk, a_bk, a_Wv, a_bv, a_Wo, a_bo,
           pe_W1, pe_b1, pe_W2, pe_b2, pn_W1, pn_b1, pn_W2, pn_b2,
           dec_W1, dec_b1, dec_W2, dec_b2):
    row = edge_index[0]
    col = edge_index[1]

    x_p = jnp.pad(x, ((0, N_PAD - N), (0, 0)))
    batch_p = jnp.pad(batch, (0, N_PAD - N))
    batch3 = batch_p.reshape(N_BLKS, 1, TILE_N)
    row_g = jnp.pad(row, (0, E_PAD - E)).reshape(E_PAD // IDX_ROW, IDX_ROW)
    col_g = jnp.pad(col, (0, E_PAD - E)).reshape(E_PAD // IDX_ROW, IDX_ROW)
    row_s = jnp.pad(row, (0, E_PAD - E),
                    constant_values=N + 16).reshape(E_PAD // IDX_ROW, IDX_ROW)
    attr_p = jnp.pad(edge_attr, ((0, E_PAD - E), (0, 0)))
    zeros_nh = jnp.zeros((BUF_ROWS, 2 * H), _f32)

    r2 = lambda b: b.reshape(1, -1)
    W1d_all = pe_W1[:, 3 * H:, :]
    peb1_all = pe_b1.reshape(L, 1, H)
    Wnt_all = pn_W1[:, 2 * H:, :]
    pnb1_all = pn_b1.reshape(L, 1, H)

    ug_all, attc_all = pl.pallas_call(
        _tables_body,
        grid=(1,),
        in_specs=[_full((B, 16)), _full((16, H)), _full((1, H)),
                  _full((H, H)), _full((1, H)),
                  _full((H, H)), _full((1, H)), _full((H, H)), _full((1, H)),
                  _full((L, H, H)), _full((L, 1, H)),
                  _full((L, H, H)), _full((L, 1, H))],
        out_specs=[_full((L, B, H)), _full((L, B, H))],
        out_shape=[jax.ShapeDtypeStruct((L, B, H), _f32),
                   jax.ShapeDtypeStruct((L, B, H), _f32)],
    )(conditions, ce_W1, r2(ce_b1), ce_W2, r2(ce_b2),
      a_Wv, r2(a_bv), a_Wo, r2(a_bo), W1d_all, peb1_all, Wnt_all, pnb1_all)

    h = pl.pallas_call(
        _encode_body,
        grid=(N_BLKS,),
        in_specs=[_row_spec(D_IN), _full((D_IN, H)), _full((1, H)),
                  _full((H, H)), _full((1, H))],
        out_specs=_row_spec(),
        out_shape=jax.ShapeDtypeStruct((N_PAD, H), _f32),
    )(x_p, ne_W1, r2(ne_b1), ne_W2, r2(ne_b2))

    ea = None
    for l in range(L):
        W1 = pe_W1[l]
        W1a, W1b, W1c = W1[:H], W1[H:2 * H], W1[2 * H:3 * H]

        tab = pl.pallas_call(
            _prep_body,
            grid=(N_BLKS,),
            in_specs=[_row_spec(), _b3_spec(), _full((B, H)),
                      _full((H, H)), _full((H, H))],
            out_specs=_row_spec(2 * H),
            out_shape=jax.ShapeDtypeStruct((N_PAD, 2 * H), _f32),
        )(h, batch3, ug_all[l], W1a, W1b)

        g1, g2 = _sc_gather2(tab, row_g, col_g)

        g_spec = pl.BlockSpec((TILE_E, 2 * H), lambda i: (i, 0))
        out_espec = pl.BlockSpec((TILE_E, 2 * H), lambda i: (i, 0))
        if l == 0:
            ea = pl.pallas_call(
                _edge0_body,
                grid=(E_BLKS,),
                in_specs=[g_spec, g_spec,
                          pl.BlockSpec((TILE_E, 16), lambda i: (i, 0)),
                          _full((16, H)), _full((1, H)),
                          _full((H, H)), _full((1, H)),
                          _full((H, H)), _full((H, H)), _full((1, H))],
                out_specs=out_espec,
                out_shape=jax.ShapeDtypeStruct((E_PAD, 2 * H), _f32),
            )(g1, g2, attr_p, ee_W1, r2(ee_b1), ee_W2, r2(ee_b2),
              W1c, pe_W2[l], r2(pe_b2[l]))
        else:
            ea = pl.pallas_call(
                _edge_body,
                grid=(E_BLKS,),
                in_specs=[g_spec, g_spec, g_spec,
                          _full((H, H)), _full((H, H)), _full((1, H))],
                out_specs=out_espec,
                out_shape=jax.ShapeDtypeStruct((E_PAD, 2 * H), _f32),
            )(g1, g2, ea, W1c, pe_W2[l], r2(pe_b2[l]))

        aggp = _sc_scatter_add(ea, row_s, zeros_nh).reshape(2, N_PAD, 2 * H)

        Wn = pn_W1[l]
        h = pl.pallas_call(
            _node_body,
            grid=(N_BLKS,),
            in_specs=[_row_spec(),
                      pl.BlockSpec((2, TILE_N, 2 * H), lambda i: (0, i, 0)),
                      _b3_spec(), _full((B, H)),
                      _full((H, H)), _full((H, H)),
                      _full((H, H)), _full((1, H))],
            out_specs=_row_spec(),
            out_shape=jax.ShapeDtypeStruct((N_PAD, H), _f32),
        )(h, aggp, batch3, attc_all[l], Wn[:H], Wn[H:2 * H],
          pn_W2[l], r2(pn_b2[l]))

    out = pl.pallas_call(
        _encode_body,
        grid=(N_BLKS,),
        in_specs=[_row_spec(), _full((H, H)), _full((1, H)),
                  _full((H, D_OUT)), _full((1, D_OUT))],
        out_specs=_row_spec(D_OUT),
        out_shape=jax.ShapeDtypeStruct((N_PAD, D_OUT), _f32),
    )(h, dec_W1, r2(dec_b1), dec_W2, r2(dec_b2))

    return out[:N]


# speedup vs baseline: 1.0044x; 1.0044x over previous
import functools

import jax
import jax.numpy as jnp
from jax import lax
from jax.experimental import pallas as pl
from jax.experimental.pallas import tpu as pltpu
from jax.experimental.pallas import tpu_sc as plsc

N = 10000
E = 320000
D_IN = 128
H = 64
B = 8
L = 3
D_OUT = 128

N_PAD = 10240
E_PAD = 327680
TILE_N = 2048
TILE_E = 2048
N_BLKS = N_PAD // TILE_N
E_BLKS = E_PAD // TILE_E

IDX_ROW = 128
ROWS_PER_TILE = (E_PAD // IDX_ROW) // 32
ROWS_C0 = 120
ROWS_C1 = 40
R_CHUNK = 2
N_CHUNKS = ROWS_PER_TILE // R_CHUNK
SLICE_N = N_PAD // 16
BUF_ROWS = 40

_f32 = jnp.float32


def _dot(a, b):
    return jnp.dot(a, b, preferred_element_type=_f32)


def _full(shape):
    return pl.BlockSpec(shape, lambda i: tuple(0 for _ in shape))


def _tables_body(cond_ref, ceW1, ceb1, ceW2, ceb2, aWv, abv, aWo, abo,
                 W1d_all, peb1_all, Wnt_all, pnb1_all, ug_out, attc_out):
    u = _dot(jnp.maximum(_dot(cond_ref[...], ceW1[...]) + ceb1[...], 0.0),
             ceW2[...]) + ceb2[...]
    att_g = _dot(_dot(u, aWv[...]) + abv[...], aWo[...]) + abo[...]
    for l in range(L):
        ug_out[l] = _dot(u, W1d_all[l]) + peb1_all[l]
        attc_out[l] = _dot(att_g, Wnt_all[l]) + pnb1_all[l]


def _encode_body(x_ref, W1, b1, W2, b2, h_ref):
    t = jnp.maximum(_dot(x_ref[...], W1[...]) + b1[...], 0.0)
    h_ref[...] = _dot(t, W2[...]) + b2[...]


def _onehot(b3_ref):
    b = b3_ref[0, 0, :]
    io = lax.broadcasted_iota(jnp.int32, (1, B), 1)
    return (b[:, None] == io).astype(_f32)


def _prep_body(h_ref, b3_ref, ug_ref, W1a, W1b, tab_ref):
    h = h_ref[...]
    oh = _onehot(b3_ref)
    ha2 = _dot(h, W1a[...]) + _dot(oh, ug_ref[...])
    hb = _dot(h, W1b[...])
    tab_ref[...] = jnp.concatenate([ha2, hb], axis=1)


def _edge0_body(g1_ref, g2_ref, attr_ref, eeW1, eeb1, eeW2, eeb2,
                W1c, W2, b2, out_ref):
    e0 = _dot(jnp.maximum(_dot(attr_ref[...], eeW1[...]) + eeb1[...], 0.0),
              eeW2[...]) + eeb2[...]
    pre = g1_ref[:, :H] + g2_ref[:, H:] + _dot(e0, W1c[...])
    ea = _dot(jnp.maximum(pre, 0.0), W2[...]) + b2[...]
    out_ref[...] = jnp.concatenate([ea, jnp.zeros_like(ea)], axis=1)


def _edge_body(g1_ref, g2_ref, ea_ref, W1c, W2, b2, out_ref):
    pre = g1_ref[:, :H] + g2_ref[:, H:] + _dot(ea_ref[:, :H], W1c[...])
    ea = _dot(jnp.maximum(pre, 0.0), W2[...]) + b2[...]
    out_ref[...] = jnp.concatenate([ea, jnp.zeros_like(ea)], axis=1)


def _node_body(h_ref, aggp_ref, b3_ref, attc_ref, Wnh, Wna, W2, b2, out_ref):
    h = h_ref[...]
    agg = aggp_ref[0, :, :H] + aggp_ref[1, :, :H]
    oh = _onehot(b3_ref)
    npre = _dot(h, Wnh[...]) + _dot(agg, Wna[...]) + _dot(oh, attc_ref[...])
    out_ref[...] = _dot(jnp.maximum(npre, 0.0), W2[...]) + b2[...] + h


def _row_spec(w=H):
    return pl.BlockSpec((TILE_N, w), lambda i: (i, 0))


def _b3_spec():
    return pl.BlockSpec((1, 1, TILE_N), lambda i: (i, 0, 0))


_MESH = plsc.VectorSubcoreMesh(core_axis_name="c", subcore_axis_name="s")


@functools.partial(
    pl.kernel,
    out_type=(jax.ShapeDtypeStruct((E_PAD, 2 * H), _f32),
              jax.ShapeDtypeStruct((E_PAD, 2 * H), _f32)),
    mesh=_MESH,
    scratch_types=[
        pltpu.VMEM((ROWS_C0, IDX_ROW), jnp.int32),
        pltpu.VMEM((ROWS_C0, IDX_ROW), jnp.int32),
        pltpu.VMEM((IDX_ROW, 2 * H), _f32),
        pltpu.VMEM((IDX_ROW, 2 * H), _f32),
        pltpu.VMEM((IDX_ROW, 2 * H), _f32),
        pltpu.VMEM((IDX_ROW, 2 * H), _f32),
        pltpu.VMEM((IDX_ROW, 2 * H), _f32),
        pltpu.VMEM((IDX_ROW, 2 * H), _f32),
        pltpu.SemaphoreType.DMA, pltpu.SemaphoreType.DMA,
        pltpu.SemaphoreType.DMA, pltpu.SemaphoreType.DMA,
        pltpu.SemaphoreType.DMA, pltpu.SemaphoreType.DMA,
        pltpu.SemaphoreType.DMA, pltpu.SemaphoreType.DMA,
        pltpu.SemaphoreType.DMA, pltpu.SemaphoreType.DMA,
        pltpu.SemaphoreType.DMA, pltpu.SemaphoreType.DMA,
    ],
)
def _sc_gather2(tab_hbm, ia_hbm, ib_hbm, ga_hbm, gb_hbm,
                ia_v, ib_v, b0, b1, b2, b3, b4, b5,
                g0, g1, g2, g3, g4, g5, w0, w1, w2, w3, w4, w5):
    sid = lax.axis_index("s")
    cid = lax.axis_index("c")
    bufs = (b0, b1, b2, b3, b4, b5)
    gsems = (g0, g1, g2, g3, g4, g5)
    wsems = (w0, w1, w2, w3, w4, w5)

    def _idx(k):
        return ia_v if k % 2 == 0 else ib_v

    def _out(k):
        return ga_hbm if k % 2 == 0 else gb_hbm

    def _wait_g(k):
        pltpu.make_async_copy(tab_hbm.at[_idx(k).at[0]], bufs[k],
                              gsems[k]).wait()

    def _wait_w(k):
        dst = _out(k).at[pl.ds(0, IDX_ROW)]
        pltpu.make_async_copy(bufs[k], dst, wsems[k]).wait()

    def _run(base_row, n_rows):
        pltpu.sync_copy(ia_hbm.at[pl.ds(base_row, n_rows)],
                        ia_v.at[pl.ds(0, n_rows)])
        pltpu.sync_copy(ib_hbm.at[pl.ds(base_row, n_rows)],
                        ib_v.at[pl.ds(0, n_rows)])

        def _fire_g(k, r):
            pltpu.async_copy(tab_hbm.at[_idx(k).at[r]], bufs[k], gsems[k])

        def _fire_w(k, r):
            dst = _out(k).at[pl.ds((base_row + r) * IDX_ROW, IDX_ROW)]
            pltpu.async_copy(bufs[k], dst, wsems[k])

        def body(i, _):
            for k in range(6):
                r = 3 * i + k // 2

                @pl.when(i > 0)
                def _():
                    _wait_w(k)

                _fire_g(k, r)
            for k in range(6):
                _wait_g(k)
                _fire_w(k, 3 * i + k // 2)
            return 0

        n_full = n_rows // 3
        lax.fori_loop(0, n_full, body, 0)
        rem = n_rows - 3 * n_full
        for t in range(rem):
            for q in range(2):
                k = 2 * t + q
                _wait_w(k)
                _fire_g(k, 3 * n_full + t)
        for t in range(rem):
            for q in range(2):
                k = 2 * t + q
                _wait_g(k)
                _fire_w(k, 3 * n_full + t)
        for k in range(6):
            _wait_w(k)

    @pl.when(cid == 0)
    def _():
        _run(sid * ROWS_C0, ROWS_C0)

    @pl.when(cid == 1)
    def _():
        _run(16 * ROWS_C0 + sid * ROWS_C1, ROWS_C1)


@functools.partial(
    pl.kernel,
    out_type=jax.ShapeDtypeStruct((2 * N_PAD, 2 * H), _f32),
    mesh=_MESH,
    scratch_types=[
        pltpu.VMEM((ROWS_PER_TILE, IDX_ROW), jnp.int32),
        pltpu.VMEM((IDX_ROW, 2 * H), _f32),
        pltpu.VMEM((IDX_ROW, 2 * H), _f32),
        pltpu.VMEM((BUF_ROWS, 2 * H), _f32),
        pltpu.VMEM_SHARED((N_PAD, 2 * H), _f32),
        pltpu.SemaphoreType.DMA, pltpu.SemaphoreType.DMA,
        pltpu.SemaphoreType.DMA, pltpu.SemaphoreType.DMA,
    ],
)
def _sc_scatter_add(ea_hbm, idx_hbm, zero_hbm, out_hbm,
                    idx_v, d0, d1, buf_v, acc_sh, l0, l1, a0, a1):
    cid = lax.axis_index("c")
    sid = lax.axis_index("s")
    wid = sid * 2 + cid
    base_row = wid * ROWS_PER_TILE
    bufs = (d0, d1)
    lsems = (l0, l1)
    asems = (a0, a1)

    pltpu.sync_copy(idx_hbm.at[pl.ds(base_row, ROWS_PER_TILE)], idx_v)
    pltpu.sync_copy(zero_hbm, buf_v)

    def zinit(p, _):
        pltpu.sync_copy(
            buf_v, acc_sh.at[pl.ds(sid * SLICE_N + p * BUF_ROWS, BUF_ROWS)])
        return 0

    lax.fori_loop(0, SLICE_N // BUF_ROWS, zinit, 0)
    plsc.subcore_barrier()

    def _fire_l(p, r):
        src = ea_hbm.at[pl.ds((base_row + r) * IDX_ROW, IDX_ROW)]
        pltpu.async_copy(src, bufs[p], lsems[p])

    def _wait_l(p):
        src = ea_hbm.at[pl.ds(0, IDX_ROW)]
        pltpu.make_async_copy(src, bufs[p], lsems[p]).wait()

    def _fire_a(p, r):
        pltpu.async_copy(bufs[p], acc_sh.at[idx_v.at[r]], asems[p], add=True)

    def _wait_a(p):
        pltpu.make_async_copy(bufs[p], acc_sh.at[idx_v.at[0]],
                              asems[p]).wait()

    def body(i, _):
        for p in range(2):
            r = 2 * i + p

            @pl.when(i > 0)
            def _():
                _wait_a(p)

            _fire_l(p, r)
        for p in range(2):
            _wait_l(p)
            _fire_a(p, 2 * i + p)
        return 0

    lax.fori_loop(0, ROWS_PER_TILE // 2, body, 0)
    for p in range(2):
        _wait_a(p)
    plsc.subcore_barrier()

    def writeout(p, _):
        r = sid * SLICE_N + p * BUF_ROWS
        pltpu.sync_copy(acc_sh.at[pl.ds(r, BUF_ROWS)], buf_v)
        pltpu.sync_copy(buf_v, out_hbm.at[pl.ds(cid * N_PAD + r, BUF_ROWS)])
        return 0

    lax.fori_loop(0, SLICE_N // BUF_ROWS, writeout, 0)


def kernel(x, edge_index, edge_attr, conditions, batch,
           ne_W1, ne_b1, ne_W2, ne_b2, ee_W1, ee_b1, ee_W2, ee_b2,
           ce_W1, ce_b1, ce_W2, ce_b2,
           a_Wq, a_bq, a_Wk, a_bk, a_Wv, a_bv, a_Wo, a_bo,
           pe_W1, pe_b1, pe_W2, pe_b2, pn_W1, pn_b1, pn_W2, pn_b2,
           dec_W1, dec_b1, dec_W2, dec_b2):
    row = edge_index[0]
    col = edge_index[1]

    x_p = jnp.pad(x, ((0, N_PAD - N), (0, 0)))
    batch_p = jnp.pad(batch, (0, N_PAD - N))
    batch3 = batch_p.reshape(N_BLKS, 1, TILE_N)
    row_g = jnp.pad(row, (0, E_PAD - E)).reshape(E_PAD // IDX_ROW, IDX_ROW)
    col_g = jnp.pad(col, (0, E_PAD - E)).reshape(E_PAD // IDX_ROW, IDX_ROW)
    row_s = jnp.pad(row, (0, E_PAD - E),
                    constant_values=N + 16).reshape(E_PAD // IDX_ROW, IDX_ROW)
    attr_p = jnp.pad(edge_attr, ((0, E_PAD - E), (0, 0)))
    zeros_nh = jnp.zeros((BUF_ROWS, 2 * H), _f32)

    r2 = lambda b: b.reshape(1, -1)
    W1d_all = pe_W1[:, 3 * H:, :]
    peb1_all = pe_b1.reshape(L, 1, H)
    Wnt_all = pn_W1[:, 2 * H:, :]
    pnb1_all = pn_b1.reshape(L, 1, H)

    ug_all, attc_all = pl.pallas_call(
        _tables_body,
        grid=(1,),
        in_specs=[_full((B, 16)), _full((16, H)), _full((1, H)),
                  _full((H, H)), _full((1, H)),
                  _full((H, H)), _full((1, H)), _full((H, H)), _full((1, H)),
                  _full((L, H, H)), _full((L, 1, H)),
                  _full((L, H, H)), _full((L, 1, H))],
        out_specs=[_full((L, B, H)), _full((L, B, H))],
        out_shape=[jax.ShapeDtypeStruct((L, B, H), _f32),
                   jax.ShapeDtypeStruct((L, B, H), _f32)],
    )(conditions, ce_W1, r2(ce_b1), ce_W2, r2(ce_b2),
      a_Wv, r2(a_bv), a_Wo, r2(a_bo), W1d_all, peb1_all, Wnt_all, pnb1_all)

    h = pl.pallas_call(
        _encode_body,
        grid=(N_BLKS,),
        in_specs=[_row_spec(D_IN), _full((D_IN, H)), _full((1, H)),
                  _full((H, H)), _full((1, H))],
        out_specs=_row_spec(),
        out_shape=jax.ShapeDtypeStruct((N_PAD, H), _f32),
    )(x_p, ne_W1, r2(ne_b1), ne_W2, r2(ne_b2))

    ea = None
    for l in range(L):
        W1 = pe_W1[l]
        W1a, W1b, W1c = W1[:H], W1[H:2 * H], W1[2 * H:3 * H]

        tab = pl.pallas_call(
            _prep_body,
            grid=(N_BLKS,),
            in_specs=[_row_spec(), _b3_spec(), _full((B, H)),
                      _full((H, H)), _full((H, H))],
            out_specs=_row_spec(2 * H),
            out_shape=jax.ShapeDtypeStruct((N_PAD, 2 * H), _f32),
        )(h, batch3, ug_all[l], W1a, W1b)

        g1, g2 = _sc_gather2(tab, row_g, col_g)

        g_spec = pl.BlockSpec((TILE_E, 2 * H), lambda i: (i, 0))
        out_espec = pl.BlockSpec((TILE_E, 2 * H), lambda i: (i, 0))
        if l == 0:
            ea = pl.pallas_call(
                _edge0_body,
                grid=(E_BLKS,),
                in_specs=[g_spec, g_spec,
                          pl.BlockSpec((TILE_E, 16), lambda i: (i, 0)),
                          _full((16, H)), _full((1, H)),
                          _full((H, H)), _full((1, H)),
                          _full((H, H)), _full((H, H)), _full((1, H))],
                out_specs=out_espec,
                out_shape=jax.ShapeDtypeStruct((E_PAD, 2 * H), _f32),
            )(g1, g2, attr_p, ee_W1, r2(ee_b1), ee_W2, r2(ee_b2),
              W1c, pe_W2[l], r2(pe_b2[l]))
        else:
            ea = pl.pallas_call(
                _edge_body,
                grid=(E_BLKS,),
                in_specs=[g_spec, g_spec, g_spec,
                          _full((H, H)), _full((H, H)), _full((1, H))],
                out_specs=out_espec,
                out_shape=jax.ShapeDtypeStruct((E_PAD, 2 * H), _f32),
            )(g1, g2, ea, W1c, pe_W2[l], r2(pe_b2[l]))

        aggp = _sc_scatter_add(ea, row_s, zeros_nh).reshape(2, N_PAD, 2 * H)

        Wn = pn_W1[l]
        h = pl.pallas_call(
            _node_body,
            grid=(N_BLKS,),
            in_specs=[_row_spec(),
                      pl.BlockSpec((2, TILE_N, 2 * H), lambda i: (0, i, 0)),
                      _b3_spec(), _full((B, H)),
                      _full((H, H)), _full((H, H)),
                      _full((H, H)), _full((1, H))],
            out_specs=_row_spec(),
            out_shape=jax.ShapeDtypeStruct((N_PAD, H), _f32),
        )(h, aggp, batch3, attc_all[l], Wn[:H], Wn[H:2 * H],
          pn_W2[l], r2(pn_b2[l]))

    out = pl.pallas_call(
        _encode_body,
        grid=(N_BLKS,),
        in_specs=[_row_spec(), _full((H, H)), _full((1, H)),
                  _full((H, D_OUT)), _full((1, D_OUT))],
        out_specs=_row_spec(D_OUT),
        out_shape=jax.ShapeDtypeStruct((N_PAD, D_OUT), _f32),
    )(h, dec_W1, r2(dec_b1), dec_W2, r2(dec_b2))

    return out[:N]


# speedup vs baseline: 1.2562x; 1.2507x over previous
import functools

import jax
import jax.numpy as jnp
from jax import lax
from jax.experimental import pallas as pl
from jax.experimental.pallas import tpu as pltpu
from jax.experimental.pallas import tpu_sc as plsc

N = 10000
E = 320000
D_IN = 128
H = 64
B = 8
L = 3
D_OUT = 128

N_PAD = 10240
E_PAD = 327680
TILE_N = 2048
TILE_E = 2048
N_BLKS = N_PAD // TILE_N
E_BLKS = E_PAD // TILE_E

IDX_ROW = 128
ROWS_PER_TILE = (E_PAD // IDX_ROW) // 32
SLICE_N = N_PAD // 16
BUF_ROWS = 40

_f32 = jnp.float32
_SC_PARAMS = pltpu.CompilerParams(use_tc_tiling_on_sc=False)


def _dot(a, b):
    return jnp.dot(a, b, preferred_element_type=_f32)


def _full(shape):
    return pl.BlockSpec(shape, lambda i: tuple(0 for _ in shape))


def _tables_body(cond_ref, ceW1, ceb1, ceW2, ceb2, aWv, abv, aWo, abo,
                 W1d_all, peb1_all, Wnt_all, pnb1_all, ug_out, attc_out):
    u = _dot(jnp.maximum(_dot(cond_ref[...], ceW1[...]) + ceb1[...], 0.0),
             ceW2[...]) + ceb2[...]
    att_g = _dot(_dot(u, aWv[...]) + abv[...], aWo[...]) + abo[...]
    for l in range(L):
        ug_out[l] = _dot(u, W1d_all[l]) + peb1_all[l]
        attc_out[l] = _dot(att_g, Wnt_all[l]) + pnb1_all[l]


def _encode_body(x_ref, W1, b1, W2, b2, h_ref):
    t = jnp.maximum(_dot(x_ref[...], W1[...]) + b1[...], 0.0)
    h_ref[...] = _dot(t, W2[...]) + b2[...]


def _onehot(b3_ref):
    b = b3_ref[0, 0, :]
    io = lax.broadcasted_iota(jnp.int32, (1, B), 1)
    return (b[:, None] == io).astype(_f32)


def _prep_body(h_ref, b3_ref, ug_ref, W1a, W1b, ha2_ref, hb_ref):
    h = h_ref[...]
    oh = _onehot(b3_ref)
    ha2_ref[...] = _dot(h, W1a[...]) + _dot(oh, ug_ref[...])
    hb_ref[...] = _dot(h, W1b[...])


def _edge0_body(ga_ref, gb_ref, attr_ref, eeW1, eeb1, eeW2, eeb2,
                W1c, W2, b2, out_ref):
    e0 = _dot(jnp.maximum(_dot(attr_ref[...], eeW1[...]) + eeb1[...], 0.0),
              eeW2[...]) + eeb2[...]
    pre = ga_ref[...] + gb_ref[...] + _dot(e0, W1c[...])
    out_ref[...] = _dot(jnp.maximum(pre, 0.0), W2[...]) + b2[...]


def _edge_body(ga_ref, gb_ref, ea_ref, W1c, W2, b2, out_ref):
    pre = ga_ref[...] + gb_ref[...] + _dot(ea_ref[...], W1c[...])
    out_ref[...] = _dot(jnp.maximum(pre, 0.0), W2[...]) + b2[...]


def _node_body(h_ref, aggp_ref, b3_ref, attc_ref, Wnh, Wna, W2, b2, out_ref):
    h = h_ref[...]
    agg = aggp_ref[0] + aggp_ref[1]
    oh = _onehot(b3_ref)
    npre = _dot(h, Wnh[...]) + _dot(agg, Wna[...]) + _dot(oh, attc_ref[...])
    out_ref[...] = _dot(jnp.maximum(npre, 0.0), W2[...]) + b2[...] + h


def _row_spec(w=H):
    return pl.BlockSpec((TILE_N, w), lambda i: (i, 0))


def _b3_spec():
    return pl.BlockSpec((1, 1, TILE_N), lambda i: (i, 0, 0))


_MESH = plsc.VectorSubcoreMesh(core_axis_name="c", subcore_axis_name="s")


@functools.partial(
    pl.kernel,
    out_type=(jax.ShapeDtypeStruct((E_PAD, H), _f32),
              jax.ShapeDtypeStruct((E_PAD, H), _f32)),
    mesh=_MESH,
    compiler_params=_SC_PARAMS,
    scratch_types=[
        pltpu.VMEM((ROWS_PER_TILE, IDX_ROW), jnp.int32),
        pltpu.VMEM((ROWS_PER_TILE, IDX_ROW), jnp.int32),
        pltpu.VMEM((IDX_ROW, H), _f32),
        pltpu.VMEM((IDX_ROW, H), _f32),
        pltpu.VMEM((IDX_ROW, H), _f32),
        pltpu.VMEM((IDX_ROW, H), _f32),
        pltpu.VMEM((IDX_ROW, H), _f32),
        pltpu.VMEM((IDX_ROW, H), _f32),
        pltpu.SemaphoreType.DMA, pltpu.SemaphoreType.DMA,
        pltpu.SemaphoreType.DMA, pltpu.SemaphoreType.DMA,
        pltpu.SemaphoreType.DMA, pltpu.SemaphoreType.DMA,
        pltpu.SemaphoreType.DMA, pltpu.SemaphoreType.DMA,
        pltpu.SemaphoreType.DMA, pltpu.SemaphoreType.DMA,
        pltpu.SemaphoreType.DMA, pltpu.SemaphoreType.DMA,
    ],
)
def _sc_gather2(ta_hbm, tb_hbm, ia_hbm, ib_hbm, ga_hbm, gb_hbm,
                ia_v, ib_v, b0, b1, b2, b3, b4, b5,
                g0, g1, g2, g3, g4, g5, w0, w1, w2, w3, w4, w5):
    sid = lax.axis_index("s")
    cid = lax.axis_index("c")
    base_row = (sid * 2 + cid) * ROWS_PER_TILE
    bufs = (b0, b1, b2, b3, b4, b5)
    gsems = (g0, g1, g2, g3, g4, g5)
    wsems = (w0, w1, w2, w3, w4, w5)

    def _idx(k):
        return ia_v if k % 2 == 0 else ib_v

    def _tab(k):
        return ta_hbm if k % 2 == 0 else tb_hbm

    def _out(k):
        return ga_hbm if k % 2 == 0 else gb_hbm

    def _wait_g(k):
        pltpu.make_async_copy(_tab(k).at[_idx(k).at[0]], bufs[k],
                              gsems[k]).wait()

    def _wait_w(k):
        dst = _out(k).at[pl.ds(0, IDX_ROW)]
        pltpu.make_async_copy(bufs[k], dst, wsems[k]).wait()

    def _fire_g(k, r):
        pltpu.async_copy(_tab(k).at[_idx(k).at[r]], bufs[k], gsems[k])

    def _fire_w(k, r):
        dst = _out(k).at[pl.ds((base_row + r) * IDX_ROW, IDX_ROW)]
        pltpu.async_copy(bufs[k], dst, wsems[k])

    pltpu.sync_copy(ia_hbm.at[pl.ds(base_row, ROWS_PER_TILE)], ia_v)
    pltpu.sync_copy(ib_hbm.at[pl.ds(base_row, ROWS_PER_TILE)], ib_v)

    def body(i, _):
        for k in range(6):
            r = 3 * i + k // 2

            @pl.when(i > 0)
            def _():
                _wait_w(k)

            _fire_g(k, r)
        for k in range(6):
            _wait_g(k)
            _fire_w(k, 3 * i + k // 2)
        return 0

    n_full = ROWS_PER_TILE // 3
    lax.fori_loop(0, n_full, body, 0)
    tail0 = 3 * n_full
    for k in range(4):
        _wait_w(k)
        _fire_g(k, tail0 + k // 2)
    for k in range(4):
        _wait_g(k)
        _fire_w(k, tail0 + k // 2)
    for k in range(6):
        _wait_w(k)


@functools.partial(
    pl.kernel,
    out_type=jax.ShapeDtypeStruct((2 * N_PAD, H), _f32),
    mesh=_MESH,
    compiler_params=_SC_PARAMS,
    scratch_types=[
        pltpu.VMEM((ROWS_PER_TILE, IDX_ROW), jnp.int32),
        pltpu.VMEM((IDX_ROW, H), _f32),
        pltpu.VMEM((IDX_ROW, H), _f32),
        pltpu.VMEM((BUF_ROWS, H), _f32),
        pltpu.VMEM_SHARED((N_PAD, H), _f32),
        pltpu.SemaphoreType.DMA, pltpu.SemaphoreType.DMA,
        pltpu.SemaphoreType.DMA, pltpu.SemaphoreType.DMA,
    ],
)
def _sc_scatter_add(ea_hbm, idx_hbm, zero_hbm, out_hbm,
                    idx_v, d0, d1, buf_v, acc_sh, l0, l1, a0, a1):
    cid = lax.axis_index("c")
    sid = lax.axis_index("s")
    base_row = (sid * 2 + cid) * ROWS_PER_TILE
    bufs = (d0, d1)
    lsems = (l0, l1)
    asems = (a0, a1)

    pltpu.sync_copy(idx_hbm.at[pl.ds(base_row, ROWS_PER_TILE)], idx_v)
    pltpu.sync_copy(zero_hbm, buf_v)

    def zinit(p, _):
        pltpu.sync_copy(
            buf_v, acc_sh.at[pl.ds(sid * SLICE_N + p * BUF_ROWS, BUF_ROWS)])
        return 0

    lax.fori_loop(0, SLICE_N // BUF_ROWS, zinit, 0)
    plsc.subcore_barrier()

    def _fire_l(p, r):
        src = ea_hbm.at[pl.ds((base_row + r) * IDX_ROW, IDX_ROW)]
        pltpu.async_copy(src, bufs[p], lsems[p])

    def _wait_l(p):
        src = ea_hbm.at[pl.ds(0, IDX_ROW)]
        pltpu.make_async_copy(src, bufs[p], lsems[p]).wait()

    def _fire_a(p, r):
        pltpu.async_copy(bufs[p], acc_sh.at[idx_v.at[r]], asems[p], add=True)

    def _wait_a(p):
        pltpu.make_async_copy(bufs[p], acc_sh.at[idx_v.at[0]],
                              asems[p]).wait()

    def body(i, _):
        for p in range(2):
            r = 2 * i + p

            @pl.when(i > 0)
            def _():
                _wait_a(p)

            _fire_l(p, r)
        for p in range(2):
            _wait_l(p)
            _fire_a(p, 2 * i + p)
        return 0

    lax.fori_loop(0, ROWS_PER_TILE // 2, body, 0)
    for p in range(2):
        _wait_a(p)
    plsc.subcore_barrier()

    def writeout(p, _):
        r = sid * SLICE_N + p * BUF_ROWS
        pltpu.sync_copy(acc_sh.at[pl.ds(r, BUF_ROWS)], buf_v)
        pltpu.sync_copy(buf_v, out_hbm.at[pl.ds(cid * N_PAD + r, BUF_ROWS)])
        return 0

    lax.fori_loop(0, SLICE_N // BUF_ROWS, writeout, 0)


def kernel(x, edge_index, edge_attr, conditions, batch,
           ne_W1, ne_b1, ne_W2, ne_b2, ee_W1, ee_b1, ee_W2, ee_b2,
           ce_W1, ce_b1, ce_W2, ce_b2,
           a_Wq, a_bq, a_Wk, a_bk, a_Wv, a_bv, a_Wo, a_bo,
           pe_W1, pe_b1, pe_W2, pe_b2, pn_W1, pn_b1, pn_W2, pn_b2,
           dec_W1, dec_b1, dec_W2, dec_b2):
    row = edge_index[0]
    col = edge_index[1]

    x_p = jnp.pad(x, ((0, N_PAD - N), (0, 0)))
    batch_p = jnp.pad(batch, (0, N_PAD - N))
    batch3 = batch_p.reshape(N_BLKS, 1, TILE_N)
    row_g = jnp.pad(row, (0, E_PAD - E)).reshape(E_PAD // IDX_ROW, IDX_ROW)
    col_g = jnp.pad(col, (0, E_PAD - E)).reshape(E_PAD // IDX_ROW, IDX_ROW)
    row_s = jnp.pad(row, (0, E_PAD - E),
                    constant_values=N + 16).reshape(E_PAD // IDX_ROW, IDX_ROW)
    attr_p = jnp.pad(edge_attr, ((0, E_PAD - E), (0, 0)))
    zeros_nh = jnp.zeros((BUF_ROWS, H), _f32)

    r2 = lambda b: b.reshape(1, -1)
    W1d_all = pe_W1[:, 3 * H:, :]
    peb1_all = pe_b1.reshape(L, 1, H)
    Wnt_all = pn_W1[:, 2 * H:, :]
    pnb1_all = pn_b1.reshape(L, 1, H)

    ug_all, attc_all = pl.pallas_call(
        _tables_body,
        grid=(1,),
        in_specs=[_full((B, 16)), _full((16, H)), _full((1, H)),
                  _full((H, H)), _full((1, H)),
                  _full((H, H)), _full((1, H)), _full((H, H)), _full((1, H)),
                  _full((L, H, H)), _full((L, 1, H)),
                  _full((L, H, H)), _full((L, 1, H))],
        out_specs=[_full((L, B, H)), _full((L, B, H))],
        out_shape=[jax.ShapeDtypeStruct((L, B, H), _f32),
                   jax.ShapeDtypeStruct((L, B, H), _f32)],
    )(conditions, ce_W1, r2(ce_b1), ce_W2, r2(ce_b2),
      a_Wv, r2(a_bv), a_Wo, r2(a_bo), W1d_all, peb1_all, Wnt_all, pnb1_all)

    h = pl.pallas_call(
        _encode_body,
        grid=(N_BLKS,),
        in_specs=[_row_spec(D_IN), _full((D_IN, H)), _full((1, H)),
                  _full((H, H)), _full((1, H))],
        out_specs=_row_spec(),
        out_shape=jax.ShapeDtypeStruct((N_PAD, H), _f32),
    )(x_p, ne_W1, r2(ne_b1), ne_W2, r2(ne_b2))

    ea = None
    for l in range(L):
        W1 = pe_W1[l]
        W1a, W1b, W1c = W1[:H], W1[H:2 * H], W1[2 * H:3 * H]

        ha2, hb = pl.pallas_call(
            _prep_body,
            grid=(N_BLKS,),
            in_specs=[_row_spec(), _b3_spec(), _full((B, H)),
                      _full((H, H)), _full((H, H))],
            out_specs=[_row_spec(), _row_spec()],
            out_shape=[jax.ShapeDtypeStruct((N_PAD, H), _f32),
                       jax.ShapeDtypeStruct((N_PAD, H), _f32)],
        )(h, batch3, ug_all[l], W1a, W1b)

        ga, gb = _sc_gather2(ha2, hb, row_g, col_g)

        espec = pl.BlockSpec((TILE_E, H), lambda i: (i, 0))
        if l == 0:
            ea = pl.pallas_call(
                _edge0_body,
                grid=(E_BLKS,),
                in_specs=[espec, espec,
                          pl.BlockSpec((TILE_E, 16), lambda i: (i, 0)),
                          _full((16, H)), _full((1, H)),
                          _full((H, H)), _full((1, H)),
                          _full((H, H)), _full((H, H)), _full((1, H))],
                out_specs=espec,
                out_shape=jax.ShapeDtypeStruct((E_PAD, H), _f32),
            )(ga, gb, attr_p, ee_W1, r2(ee_b1), ee_W2, r2(ee_b2),
              W1c, pe_W2[l], r2(pe_b2[l]))
        else:
            ea = pl.pallas_call(
                _edge_body,
                grid=(E_BLKS,),
                in_specs=[espec, espec, espec,
                          _full((H, H)), _full((H, H)), _full((1, H))],
                out_specs=espec,
                out_shape=jax.ShapeDtypeStruct((E_PAD, H), _f32),
            )(ga, gb, ea, W1c, pe_W2[l], r2(pe_b2[l]))

        aggp = _sc_scatter_add(ea, row_s, zeros_nh).reshape(2, N_PAD, H)

        Wn = pn_W1[l]
        h = pl.pallas_call(
            _node_body,
            grid=(N_BLKS,),
            in_specs=[_row_spec(),
                      pl.BlockSpec((2, TILE_N, H), lambda i: (0, i, 0)),
                      _b3_spec(), _full((B, H)),
                      _full((H, H)), _full((H, H)),
                      _full((H, H)), _full((1, H))],
            out_specs=_row_spec(),
            out_shape=jax.ShapeDtypeStruct((N_PAD, H), _f32),
        )(h, aggp, batch3, attc_all[l], Wn[:H], Wn[H:2 * H],
          pn_W2[l], r2(pn_b2[l]))

    out = pl.pallas_call(
        _encode_body,
        grid=(N_BLKS,),
        in_specs=[_row_spec(), _full((H, H)), _full((1, H)),
                  _full((H, D_OUT)), _full((1, D_OUT))],
        out_specs=_row_spec(D_OUT),
        out_shape=jax.ShapeDtypeStruct((N_PAD, D_OUT), _f32),
    )(h, dec_W1, r2(dec_b1), dec_W2, r2(dec_b2))

    return out[:N]


# speedup vs baseline: 1.9764x; 1.5733x over previous
import functools

import jax
import jax.numpy as jnp
from jax import lax
from jax.experimental import pallas as pl
from jax.experimental.pallas import tpu as pltpu
from jax.experimental.pallas import tpu_sc as plsc

N = 10000
E = 320000
D_IN = 128
H = 64
B = 8
L = 3
D_OUT = 128

N_PAD = 10240
E_PAD = 327680
TILE_N = 2048
TILE_E = 2048
N_BLKS = N_PAD // TILE_N
E_BLKS = E_PAD // TILE_E

IDX_ROW = 128
ROWS_PER_TILE = (E_PAD // IDX_ROW) // 32
SLICE_N = N_PAD // 16
BUF_ROWS = 40

_f32 = jnp.float32
_SC_PARAMS = pltpu.CompilerParams(use_tc_tiling_on_sc=False)


def _dot(a, b):
    return jnp.dot(a, b, preferred_element_type=_f32)


def _full(shape):
    return pl.BlockSpec(shape, lambda i: tuple(0 for _ in shape))


def _tables_body(cond_ref, ceW1, ceb1, ceW2, ceb2, aWv, abv, aWo, abo,
                 W1d_all, peb1_all, Wnt_all, pnb1_all, ug_out, attc_out):
    u = _dot(jnp.maximum(_dot(cond_ref[...], ceW1[...]) + ceb1[...], 0.0),
             ceW2[...]) + ceb2[...]
    att_g = _dot(_dot(u, aWv[...]) + abv[...], aWo[...]) + abo[...]
    for l in range(L):
        ug_out[l] = _dot(u, W1d_all[l]) + peb1_all[l]
        attc_out[l] = _dot(att_g, Wnt_all[l]) + pnb1_all[l]


def _encode_body(x_ref, W1, b1, W2, b2, h_ref):
    t = jnp.maximum(_dot(x_ref[...], W1[...]) + b1[...], 0.0)
    h_ref[...] = _dot(t, W2[...]) + b2[...]


def _onehot(b3_ref):
    b = b3_ref[0, 0, :]
    io = lax.broadcasted_iota(jnp.int32, (1, B), 1)
    return (b[:, None] == io).astype(_f32)


def _prep_body(h_ref, b3_ref, ug_ref, W1a, W1b, ha2_ref, hb_ref):
    h = h_ref[...]
    oh = _onehot(b3_ref)
    ha2_ref[...] = _dot(h, W1a[...]) + _dot(oh, ug_ref[...])
    hb_ref[...] = _dot(h, W1b[...])


def _edge0_body(ga_ref, gb_ref, attr_ref, eeW1, eeb1, eeW2, eeb2,
                W1c, W2, b2, out_ref):
    e0 = _dot(jnp.maximum(_dot(attr_ref[...], eeW1[...]) + eeb1[...], 0.0),
              eeW2[...]) + eeb2[...]
    pre = ga_ref[...] + gb_ref[...] + _dot(e0, W1c[...])
    out_ref[...] = _dot(jnp.maximum(pre, 0.0), W2[...]) + b2[...]


def _edge_body(ga_ref, gb_ref, ea_ref, W1c, W2, b2, out_ref):
    pre = ga_ref[...] + gb_ref[...] + _dot(ea_ref[...], W1c[...])
    out_ref[...] = _dot(jnp.maximum(pre, 0.0), W2[...]) + b2[...]


def _node_body(h_ref, aggp_ref, b3_ref, attc_ref, Wnh, Wna, W2, b2, out_ref):
    h = h_ref[...]
    agg = aggp_ref[0] + aggp_ref[1]
    oh = _onehot(b3_ref)
    npre = _dot(h, Wnh[...]) + _dot(agg, Wna[...]) + _dot(oh, attc_ref[...])
    out_ref[...] = _dot(jnp.maximum(npre, 0.0), W2[...]) + b2[...] + h


def _row_spec(w=H):
    return pl.BlockSpec((TILE_N, w), lambda i: (i, 0))


def _b3_spec():
    return pl.BlockSpec((1, 1, TILE_N), lambda i: (i, 0, 0))


_MESH = plsc.VectorSubcoreMesh(core_axis_name="c", subcore_axis_name="s")


@functools.partial(
    pl.kernel,
    out_type=(jax.ShapeDtypeStruct((E_PAD, H), _f32),
              jax.ShapeDtypeStruct((E_PAD, H), _f32)),
    mesh=_MESH,
    compiler_params=_SC_PARAMS,
    scratch_types=[
        pltpu.VMEM((ROWS_PER_TILE, IDX_ROW), jnp.int32),
        pltpu.VMEM((ROWS_PER_TILE, IDX_ROW), jnp.int32),
        pltpu.VMEM((IDX_ROW, H), _f32),
        pltpu.VMEM((IDX_ROW, H), _f32),
        pltpu.VMEM((IDX_ROW, H), _f32),
        pltpu.VMEM((IDX_ROW, H), _f32),
        pltpu.VMEM((IDX_ROW, H), _f32),
        pltpu.VMEM((IDX_ROW, H), _f32),
        pltpu.SemaphoreType.DMA, pltpu.SemaphoreType.DMA,
        pltpu.SemaphoreType.DMA, pltpu.SemaphoreType.DMA,
        pltpu.SemaphoreType.DMA, pltpu.SemaphoreType.DMA,
        pltpu.SemaphoreType.DMA, pltpu.SemaphoreType.DMA,
        pltpu.SemaphoreType.DMA, pltpu.SemaphoreType.DMA,
        pltpu.SemaphoreType.DMA, pltpu.SemaphoreType.DMA,
    ],
)
def _sc_gather2(ta_hbm, tb_hbm, ia_hbm, ib_hbm, ga_hbm, gb_hbm,
                ia_v, ib_v, b0, b1, b2, b3, b4, b5,
                g0, g1, g2, g3, g4, g5, w0, w1, w2, w3, w4, w5):
    sid = lax.axis_index("s")
    cid = lax.axis_index("c")
    base_row = (sid * 2 + cid) * ROWS_PER_TILE
    bufs = (b0, b1, b2, b3, b4, b5)
    gsems = (g0, g1, g2, g3, g4, g5)
    wsems = (w0, w1, w2, w3, w4, w5)

    def _idx(k):
        return ia_v if k % 2 == 0 else ib_v

    def _tab(k):
        return ta_hbm if k % 2 == 0 else tb_hbm

    def _out(k):
        return ga_hbm if k % 2 == 0 else gb_hbm

    def _wait_g(k):
        pltpu.make_async_copy(_tab(k).at[_idx(k).at[0]], bufs[k],
                              gsems[k]).wait()

    def _wait_w(k):
        dst = _out(k).at[pl.ds(0, IDX_ROW)]
        pltpu.make_async_copy(bufs[k], dst, wsems[k]).wait()

    def _fire_g(k, r):
        pltpu.async_copy(_tab(k).at[_idx(k).at[r]], bufs[k], gsems[k])

    def _fire_w(k, r):
        dst = _out(k).at[pl.ds((base_row + r) * IDX_ROW, IDX_ROW)]
        pltpu.async_copy(bufs[k], dst, wsems[k])

    pltpu.sync_copy(ia_hbm.at[pl.ds(base_row, ROWS_PER_TILE)], ia_v)
    pltpu.sync_copy(ib_hbm.at[pl.ds(base_row, ROWS_PER_TILE)], ib_v)

    def body(i, _):
        for k in range(6):
            r = 3 * i + k // 2

            @pl.when(i > 0)
            def _():
                _wait_w(k)

            _fire_g(k, r)
        for k in range(6):
            _wait_g(k)
            _fire_w(k, 3 * i + k // 2)
        return 0

    n_full = ROWS_PER_TILE // 3
    lax.fori_loop(0, n_full, body, 0)
    tail0 = 3 * n_full
    for k in range(4):
        _wait_w(k)
        _fire_g(k, tail0 + k // 2)
    for k in range(4):
        _wait_g(k)
        _fire_w(k, tail0 + k // 2)
    for k in range(6):
        _wait_w(k)


@functools.partial(
    pl.kernel,
    out_type=jax.ShapeDtypeStruct((2 * N_PAD, H), _f32),
    mesh=_MESH,
    compiler_params=_SC_PARAMS,
    scratch_types=[
        pltpu.VMEM((ROWS_PER_TILE, IDX_ROW), jnp.int32),
        pltpu.VMEM((IDX_ROW, H), _f32),
        pltpu.VMEM((IDX_ROW, H), _f32),
        pltpu.VMEM((BUF_ROWS, H), _f32),
        pltpu.VMEM_SHARED((N_PAD, H), _f32),
        pltpu.SemaphoreType.DMA, pltpu.SemaphoreType.DMA,
        pltpu.SemaphoreType.DMA, pltpu.SemaphoreType.DMA,
    ],
)
def _sc_scatter_add(ea_hbm, idx_hbm, zero_hbm, out_hbm,
                    idx_v, d0, d1, buf_v, acc_sh, l0, l1, a0, a1):
    cid = lax.axis_index("c")
    sid = lax.axis_index("s")
    base_row = (sid * 2 + cid) * ROWS_PER_TILE
    bufs = (d0, d1)
    lsems = (l0, l1)
    asems = (a0, a1)

    pltpu.sync_copy(idx_hbm.at[pl.ds(base_row, ROWS_PER_TILE)], idx_v)
    pltpu.sync_copy(zero_hbm, buf_v)

    def zinit(p, _):
        pltpu.sync_copy(
            buf_v, acc_sh.at[pl.ds(sid * SLICE_N + p * BUF_ROWS, BUF_ROWS)])
        return 0

    lax.fori_loop(0, SLICE_N // BUF_ROWS, zinit, 0)
    plsc.subcore_barrier()

    def _fire_l(p, r):
        src = ea_hbm.at[pl.ds((base_row + r) * IDX_ROW, IDX_ROW)]
        pltpu.async_copy(src, bufs[p], lsems[p])

    def _wait_l(p):
        src = ea_hbm.at[pl.ds(0, IDX_ROW)]
        pltpu.make_async_copy(src, bufs[p], lsems[p]).wait()

    def _fire_a(p, r):
        pltpu.async_copy(bufs[p], acc_sh.at[idx_v.at[r]], asems[p], add=True)

    def _wait_a(p):
        pltpu.make_async_copy(bufs[p], acc_sh.at[idx_v.at[0]],
                              asems[p]).wait()

    def body(i, _):
        for p in range(2):
            r = 2 * i + p

            @pl.when(i > 0)
            def _():
                _wait_a(p)

            _fire_l(p, r)
        for p in range(2):
            _wait_l(p)
            _fire_a(p, 2 * i + p)
        return 0

    lax.fori_loop(0, ROWS_PER_TILE // 2, body, 0)
    for p in range(2):
        _wait_a(p)
    plsc.subcore_barrier()

    def writeout(p, _):
        r = sid * SLICE_N + p * BUF_ROWS
        pltpu.sync_copy(acc_sh.at[pl.ds(r, BUF_ROWS)], buf_v)
        pltpu.sync_copy(buf_v, out_hbm.at[pl.ds(cid * N_PAD + r, BUF_ROWS)])
        return 0

    lax.fori_loop(0, SLICE_N // BUF_ROWS, writeout, 0)


def kernel(x, edge_index, edge_attr, conditions, batch,
           ne_W1, ne_b1, ne_W2, ne_b2, ee_W1, ee_b1, ee_W2, ee_b2,
           ce_W1, ce_b1, ce_W2, ce_b2,
           a_Wq, a_bq, a_Wk, a_bk, a_Wv, a_bv, a_Wo, a_bo,
           pe_W1, pe_b1, pe_W2, pe_b2, pn_W1, pn_b1, pn_W2, pn_b2,
           dec_W1, dec_b1, dec_W2, dec_b2):
    row = edge_index[0]
    col = edge_index[1]

    x_p = jnp.pad(x, ((0, N_PAD - N), (0, 0)))
    batch_p = jnp.pad(batch, (0, N_PAD - N))
    batch3 = batch_p.reshape(N_BLKS, 1, TILE_N)
    row_g = jnp.pad(row, (0, E_PAD - E)).reshape(E_PAD // IDX_ROW, IDX_ROW)
    col_g = jnp.pad(col, (0, E_PAD - E)).reshape(E_PAD // IDX_ROW, IDX_ROW)
    row_s = jnp.pad(row, (0, E_PAD - E),
                    constant_values=N + 16).reshape(E_PAD // IDX_ROW, IDX_ROW)
    attr_p = jnp.pad(edge_attr, ((0, E_PAD - E), (0, 0)))
    attr2 = attr_p.reshape(E_PAD // 2, 32)
    zeros_nh = jnp.zeros((BUF_ROWS, H), _f32)

    r2 = lambda b: b.reshape(1, -1)

    def bd2(W):
        z = jnp.zeros_like(W)
        return jnp.concatenate(
            [jnp.concatenate([W, z], axis=1),
             jnp.concatenate([z, W], axis=1)], axis=0)

    def b2x(b):
        return jnp.concatenate([b, b]).reshape(1, -1)
    W1d_all = pe_W1[:, 3 * H:, :]
    peb1_all = pe_b1.reshape(L, 1, H)
    Wnt_all = pn_W1[:, 2 * H:, :]
    pnb1_all = pn_b1.reshape(L, 1, H)

    ug_all, attc_all = pl.pallas_call(
        _tables_body,
        grid=(1,),
        in_specs=[_full((B, 16)), _full((16, H)), _full((1, H)),
                  _full((H, H)), _full((1, H)),
                  _full((H, H)), _full((1, H)), _full((H, H)), _full((1, H)),
                  _full((L, H, H)), _full((L, 1, H)),
                  _full((L, H, H)), _full((L, 1, H))],
        out_specs=[_full((L, B, H)), _full((L, B, H))],
        out_shape=[jax.ShapeDtypeStruct((L, B, H), _f32),
                   jax.ShapeDtypeStruct((L, B, H), _f32)],
    )(conditions, ce_W1, r2(ce_b1), ce_W2, r2(ce_b2),
      a_Wv, r2(a_bv), a_Wo, r2(a_bo), W1d_all, peb1_all, Wnt_all, pnb1_all)

    h = pl.pallas_call(
        _encode_body,
        grid=(N_BLKS,),
        in_specs=[_row_spec(D_IN), _full((D_IN, H)), _full((1, H)),
                  _full((H, H)), _full((1, H))],
        out_specs=_row_spec(),
        out_shape=jax.ShapeDtypeStruct((N_PAD, H), _f32),
    )(x_p, ne_W1, r2(ne_b1), ne_W2, r2(ne_b2))

    ea = None
    for l in range(L):
        W1 = pe_W1[l]
        W1a, W1b, W1c = W1[:H], W1[H:2 * H], W1[2 * H:3 * H]

        ha2, hb = pl.pallas_call(
            _prep_body,
            grid=(N_BLKS,),
            in_specs=[_row_spec(), _b3_spec(), _full((B, H)),
                      _full((H, H)), _full((H, H))],
            out_specs=[_row_spec(), _row_spec()],
            out_shape=[jax.ShapeDtypeStruct((N_PAD, H), _f32),
                       jax.ShapeDtypeStruct((N_PAD, H), _f32)],
        )(h, batch3, ug_all[l], W1a, W1b)

        ga, gb = _sc_gather2(ha2, hb, row_g, col_g)
        ga = ga.reshape(E_PAD // 2, 2 * H)
        gb = gb.reshape(E_PAD // 2, 2 * H)

        espec = pl.BlockSpec((TILE_E // 2, 2 * H), lambda i: (i, 0))
        if l == 0:
            ea = pl.pallas_call(
                _edge0_body,
                grid=(E_BLKS,),
                in_specs=[espec, espec,
                          pl.BlockSpec((TILE_E // 2, 32), lambda i: (i, 0)),
                          _full((32, 2 * H)), _full((1, 2 * H)),
                          _full((2 * H, 2 * H)), _full((1, 2 * H)),
                          _full((2 * H, 2 * H)), _full((2 * H, 2 * H)),
                          _full((1, 2 * H))],
                out_specs=espec,
                out_shape=jax.ShapeDtypeStruct((E_PAD // 2, 2 * H), _f32),
            )(ga, gb, attr2, bd2(ee_W1), b2x(ee_b1), bd2(ee_W2), b2x(ee_b2),
              bd2(W1c), bd2(pe_W2[l]), b2x(pe_b2[l]))
        else:
            ea = pl.pallas_call(
                _edge_body,
                grid=(E_BLKS,),
                in_specs=[espec, espec, espec,
                          _full((2 * H, 2 * H)), _full((2 * H, 2 * H)),
                          _full((1, 2 * H))],
                out_specs=espec,
                out_shape=jax.ShapeDtypeStruct((E_PAD // 2, 2 * H), _f32),
            )(ga, gb, ea, bd2(W1c), bd2(pe_W2[l]), b2x(pe_b2[l]))

        aggp = _sc_scatter_add(ea.reshape(E_PAD, H), row_s,
                               zeros_nh).reshape(2, N_PAD, H)

        Wn = pn_W1[l]
        h = pl.pallas_call(
            _node_body,
            grid=(N_BLKS,),
            in_specs=[_row_spec(),
                      pl.BlockSpec((2, TILE_N, H), lambda i: (0, i, 0)),
                      _b3_spec(), _full((B, H)),
                      _full((H, H)), _full((H, H)),
                      _full((H, H)), _full((1, H))],
            out_specs=_row_spec(),
            out_shape=jax.ShapeDtypeStruct((N_PAD, H), _f32),
        )(h, aggp, batch3, attc_all[l], Wn[:H], Wn[H:2 * H],
          pn_W2[l], r2(pn_b2[l]))

    out = pl.pallas_call(
        _encode_body,
        grid=(N_BLKS,),
        in_specs=[_row_spec(), _full((H, H)), _full((1, H)),
                  _full((H, D_OUT)), _full((1, D_OUT))],
        out_specs=_row_spec(D_OUT),
        out_shape=jax.ShapeDtypeStruct((N_PAD, D_OUT), _f32),
    )(h, dec_W1, r2(dec_b1), dec_W2, r2(dec_b2))

    return out[:N]


# speedup vs baseline: 2.0070x; 1.0155x over previous
import functools

import jax
import jax.numpy as jnp
from jax import lax
from jax.experimental import pallas as pl
from jax.experimental.pallas import tpu as pltpu
from jax.experimental.pallas import tpu_sc as plsc

N = 10000
E = 320000
D_IN = 128
H = 64
B = 8
L = 3
D_OUT = 128

N_PAD = 10240
E_PAD = 327680
TILE_N = 2048
TILE_E = 2048
N_BLKS = N_PAD // TILE_N
E_BLKS = E_PAD // TILE_E

IDX_ROW = 128
ROWS_PER_TILE = (E_PAD // IDX_ROW) // 32
SLICE_N = N_PAD // 16
BUF_ROWS = 40

_f32 = jnp.float32
_SC_PARAMS = pltpu.CompilerParams(use_tc_tiling_on_sc=False)


def _dot(a, b):
    return jnp.dot(a, b, preferred_element_type=_f32)


def _full(shape):
    return pl.BlockSpec(shape, lambda i: tuple(0 for _ in shape))


def _tables_body(cond_ref, ceW1, ceb1, ceW2, ceb2, aWv, abv, aWo, abo,
                 W1d_all, peb1_all, Wnt_all, pnb1_all, ug_out, attc_out):
    u = _dot(jnp.maximum(_dot(cond_ref[...], ceW1[...]) + ceb1[...], 0.0),
             ceW2[...]) + ceb2[...]
    att_g = _dot(_dot(u, aWv[...]) + abv[...], aWo[...]) + abo[...]
    for l in range(L):
        ug_out[l] = _dot(u, W1d_all[l]) + peb1_all[l]
        attc_out[l] = _dot(att_g, Wnt_all[l]) + pnb1_all[l]


def _encode_body(x_ref, W1, b1, W2, b2, h_ref):
    t = jnp.maximum(_dot(x_ref[...], W1[...]) + b1[...], 0.0)
    h_ref[...] = _dot(t, W2[...]) + b2[...]


def _onehot(b3_ref):
    b = b3_ref[0, 0, :]
    io = lax.broadcasted_iota(jnp.int32, (1, B), 1)
    return (b[:, None] == io).astype(_f32)


def _prep_body(h_ref, b3_ref, ug_ref, W1a, W1b, ha2_ref, hb_ref):
    h = h_ref[...]
    oh = _onehot(b3_ref)
    ha2_ref[...] = _dot(h, W1a[...]) + _dot(oh, ug_ref[...])
    hb_ref[...] = _dot(h, W1b[...])


def _edge0_body(ga_ref, gb_ref, attr_ref, eeW1, eeb1, eeW2, eeb2,
                W1c, W2, b2, out_ref):
    e0 = _dot(jnp.maximum(_dot(attr_ref[...], eeW1[...]) + eeb1[...], 0.0),
              eeW2[...]) + eeb2[...]
    pre = ga_ref[...] + gb_ref[...] + _dot(e0, W1c[...])
    out_ref[...] = _dot(jnp.maximum(pre, 0.0), W2[...]) + b2[...]


def _edge_body(ga_ref, gb_ref, ea_ref, W1c, W2, b2, out_ref):
    pre = ga_ref[...] + gb_ref[...] + _dot(ea_ref[...], W1c[...])
    out_ref[...] = _dot(jnp.maximum(pre, 0.0), W2[...]) + b2[...]


def _node_body(h_ref, aggp_ref, b3_ref, attc_ref, Wnh, Wna, W2, b2, out_ref):
    h = h_ref[...]
    agg = aggp_ref[0] + aggp_ref[1]
    oh = _onehot(b3_ref)
    npre = _dot(h, Wnh[...]) + _dot(agg, Wna[...]) + _dot(oh, attc_ref[...])
    out_ref[...] = _dot(jnp.maximum(npre, 0.0), W2[...]) + b2[...] + h


def _row_spec(w=H):
    return pl.BlockSpec((TILE_N, w), lambda i: (i, 0))


def _b3_spec():
    return pl.BlockSpec((1, 1, TILE_N), lambda i: (i, 0, 0))


_MESH = plsc.VectorSubcoreMesh(core_axis_name="c", subcore_axis_name="s")


@functools.partial(
    pl.kernel,
    out_type=(jax.ShapeDtypeStruct((E_PAD, H), _f32),
              jax.ShapeDtypeStruct((E_PAD, H), _f32)),
    mesh=_MESH,
    compiler_params=_SC_PARAMS,
    scratch_types=[
        pltpu.VMEM((ROWS_PER_TILE, IDX_ROW), jnp.int32),
        pltpu.VMEM((ROWS_PER_TILE, IDX_ROW), jnp.int32),
        pltpu.VMEM((IDX_ROW, H), _f32),
        pltpu.VMEM((IDX_ROW, H), _f32),
        pltpu.VMEM((IDX_ROW, H), _f32),
        pltpu.VMEM((IDX_ROW, H), _f32),
        pltpu.VMEM((IDX_ROW, H), _f32),
        pltpu.VMEM((IDX_ROW, H), _f32),
        pltpu.SemaphoreType.DMA, pltpu.SemaphoreType.DMA,
        pltpu.SemaphoreType.DMA, pltpu.SemaphoreType.DMA,
        pltpu.SemaphoreType.DMA, pltpu.SemaphoreType.DMA,
        pltpu.SemaphoreType.DMA, pltpu.SemaphoreType.DMA,
        pltpu.SemaphoreType.DMA, pltpu.SemaphoreType.DMA,
        pltpu.SemaphoreType.DMA, pltpu.SemaphoreType.DMA,
    ],
)
def _sc_gather2(ta_hbm, tb_hbm, ia_hbm, ib_hbm, ga_hbm, gb_hbm,
                ia_v, ib_v, b0, b1, b2, b3, b4, b5,
                g0, g1, g2, g3, g4, g5, w0, w1, w2, w3, w4, w5):
    sid = lax.axis_index("s")
    cid = lax.axis_index("c")
    base_row = (sid * 2 + cid) * ROWS_PER_TILE
    bufs = (b0, b1, b2, b3, b4, b5)
    gsems = (g0, g1, g2, g3, g4, g5)
    wsems = (w0, w1, w2, w3, w4, w5)

    def _idx(k):
        return ia_v if k % 2 == 0 else ib_v

    def _tab(k):
        return ta_hbm if k % 2 == 0 else tb_hbm

    def _out(k):
        return ga_hbm if k % 2 == 0 else gb_hbm

    def _wait_g(k):
        pltpu.make_async_copy(_tab(k).at[_idx(k).at[0]], bufs[k],
                              gsems[k]).wait()

    def _wait_w(k):
        dst = _out(k).at[pl.ds(0, IDX_ROW)]
        pltpu.make_async_copy(bufs[k], dst, wsems[k]).wait()

    def _fire_g(k, r):
        pltpu.async_copy(_tab(k).at[_idx(k).at[r]], bufs[k], gsems[k])

    def _fire_w(k, r):
        dst = _out(k).at[pl.ds((base_row + r) * IDX_ROW, IDX_ROW)]
        pltpu.async_copy(bufs[k], dst, wsems[k])

    pltpu.sync_copy(ia_hbm.at[pl.ds(base_row, ROWS_PER_TILE)], ia_v)
    pltpu.sync_copy(ib_hbm.at[pl.ds(base_row, ROWS_PER_TILE)], ib_v)

    def body(i, _):
        for k in range(6):
            r = 3 * i + k // 2

            @pl.when(i > 0)
            def _():
                _wait_w(k)

            _fire_g(k, r)
        for k in range(6):
            _wait_g(k)
            _fire_w(k, 3 * i + k // 2)
        return 0

    n_full = ROWS_PER_TILE // 3
    lax.fori_loop(0, n_full, body, 0)
    tail0 = 3 * n_full
    for k in range(4):
        _wait_w(k)
        _fire_g(k, tail0 + k // 2)
    for k in range(4):
        _wait_g(k)
        _fire_w(k, tail0 + k // 2)
    for k in range(6):
        _wait_w(k)


@functools.partial(
    pl.kernel,
    out_type=jax.ShapeDtypeStruct((2 * N_PAD, H), _f32),
    mesh=_MESH,
    compiler_params=_SC_PARAMS,
    scratch_types=[
        pltpu.VMEM((ROWS_PER_TILE, IDX_ROW), jnp.int32),
        pltpu.VMEM((IDX_ROW, H), _f32),
        pltpu.VMEM((IDX_ROW, H), _f32),
        pltpu.VMEM((BUF_ROWS, H), _f32),
        pltpu.VMEM_SHARED((N_PAD, H), _f32),
        pltpu.SemaphoreType.DMA, pltpu.SemaphoreType.DMA,
        pltpu.SemaphoreType.DMA, pltpu.SemaphoreType.DMA,
    ],
)
def _sc_scatter_add(ea_hbm, idx_hbm, zero_hbm, out_hbm,
                    idx_v, d0, d1, buf_v, acc_sh, l0, l1, a0, a1):
    cid = lax.axis_index("c")
    sid = lax.axis_index("s")
    base_row = (sid * 2 + cid) * ROWS_PER_TILE
    bufs = (d0, d1)
    lsems = (l0, l1)
    asems = (a0, a1)

    pltpu.sync_copy(idx_hbm.at[pl.ds(base_row, ROWS_PER_TILE)], idx_v)
    pltpu.sync_copy(zero_hbm, buf_v)

    def zinit(p, _):
        pltpu.sync_copy(
            buf_v, acc_sh.at[pl.ds(sid * SLICE_N + p * BUF_ROWS, BUF_ROWS)])
        return 0

    lax.fori_loop(0, SLICE_N // BUF_ROWS, zinit, 0)
    plsc.subcore_barrier()

    def _fire_l(p, r):
        src = ea_hbm.at[pl.ds((base_row + r) * IDX_ROW, IDX_ROW)]
        pltpu.async_copy(src, bufs[p], lsems[p])

    def _wait_l(p):
        src = ea_hbm.at[pl.ds(0, IDX_ROW)]
        pltpu.make_async_copy(src, bufs[p], lsems[p]).wait()

    def _fire_a(p, r):
        pltpu.async_copy(bufs[p], acc_sh.at[idx_v.at[r]], asems[p], add=True)

    def _wait_a(p):
        pltpu.make_async_copy(bufs[p], acc_sh.at[idx_v.at[0]],
                              asems[p]).wait()

    def body(i, _):
        for p in range(2):
            r = 2 * i + p

            @pl.when(i > 0)
            def _():
                _wait_a(p)

            _fire_l(p, r)
        for p in range(2):
            _wait_l(p)
            _fire_a(p, 2 * i + p)
        return 0

    lax.fori_loop(0, ROWS_PER_TILE // 2, body, 0)
    for p in range(2):
        _wait_a(p)
    plsc.subcore_barrier()

    def writeout(p, _):
        r = sid * SLICE_N + p * BUF_ROWS
        pltpu.sync_copy(acc_sh.at[pl.ds(r, BUF_ROWS)], buf_v)
        pltpu.sync_copy(buf_v, out_hbm.at[pl.ds(cid * N_PAD + r, BUF_ROWS)])
        return 0

    lax.fori_loop(0, SLICE_N // BUF_ROWS, writeout, 0)


def kernel(x, edge_index, edge_attr, conditions, batch,
           ne_W1, ne_b1, ne_W2, ne_b2, ee_W1, ee_b1, ee_W2, ee_b2,
           ce_W1, ce_b1, ce_W2, ce_b2,
           a_Wq, a_bq, a_Wk, a_bk, a_Wv, a_bv, a_Wo, a_bo,
           pe_W1, pe_b1, pe_W2, pe_b2, pn_W1, pn_b1, pn_W2, pn_b2,
           dec_W1, dec_b1, dec_W2, dec_b2):
    row = edge_index[0]
    col = edge_index[1]

    x_p = jnp.pad(x, ((0, N_PAD - N), (0, 0)))
    batch_p = jnp.pad(batch, (0, N_PAD - N))
    batch3 = batch_p.reshape(N_BLKS, 1, TILE_N)
    row_g = jnp.pad(row, (0, E_PAD - E)).reshape(E_PAD // IDX_ROW, IDX_ROW)
    col_g = jnp.pad(col, (0, E_PAD - E)).reshape(E_PAD // IDX_ROW, IDX_ROW)
    row_s = jnp.pad(row, (0, E_PAD - E),
                    constant_values=N + 16).reshape(E_PAD // IDX_ROW, IDX_ROW)
    attr2 = jnp.pad(edge_attr.reshape(E // 2, 32),
                    ((0, (E_PAD - E) // 2), (0, 0)))
    zeros_nh = jnp.zeros((BUF_ROWS, H), _f32)

    r2 = lambda b: b.reshape(1, -1)

    def bd2(W):
        z = jnp.zeros_like(W)
        return jnp.concatenate(
            [jnp.concatenate([W, z], axis=1),
             jnp.concatenate([z, W], axis=1)], axis=0)

    def b2x(b):
        return jnp.concatenate([b, b]).reshape(1, -1)
    W1d_all = pe_W1[:, 3 * H:, :]
    peb1_all = pe_b1.reshape(L, 1, H)
    Wnt_all = pn_W1[:, 2 * H:, :]
    pnb1_all = pn_b1.reshape(L, 1, H)

    ug_all, attc_all = pl.pallas_call(
        _tables_body,
        grid=(1,),
        in_specs=[_full((B, 16)), _full((16, H)), _full((1, H)),
                  _full((H, H)), _full((1, H)),
                  _full((H, H)), _full((1, H)), _full((H, H)), _full((1, H)),
                  _full((L, H, H)), _full((L, 1, H)),
                  _full((L, H, H)), _full((L, 1, H))],
        out_specs=[_full((L, B, H)), _full((L, B, H))],
        out_shape=[jax.ShapeDtypeStruct((L, B, H), _f32),
                   jax.ShapeDtypeStruct((L, B, H), _f32)],
    )(conditions, ce_W1, r2(ce_b1), ce_W2, r2(ce_b2),
      a_Wv, r2(a_bv), a_Wo, r2(a_bo), W1d_all, peb1_all, Wnt_all, pnb1_all)

    h = pl.pallas_call(
        _encode_body,
        grid=(N_BLKS,),
        in_specs=[_row_spec(D_IN), _full((D_IN, H)), _full((1, H)),
                  _full((H, H)), _full((1, H))],
        out_specs=_row_spec(),
        out_shape=jax.ShapeDtypeStruct((N_PAD, H), _f32),
    )(x_p, ne_W1, r2(ne_b1), ne_W2, r2(ne_b2))

    ea = None
    for l in range(L):
        W1 = pe_W1[l]
        W1a, W1b, W1c = W1[:H], W1[H:2 * H], W1[2 * H:3 * H]

        ha2, hb = pl.pallas_call(
            _prep_body,
            grid=(N_BLKS,),
            in_specs=[_row_spec(), _b3_spec(), _full((B, H)),
                      _full((H, H)), _full((H, H))],
            out_specs=[_row_spec(), _row_spec()],
            out_shape=[jax.ShapeDtypeStruct((N_PAD, H), _f32),
                       jax.ShapeDtypeStruct((N_PAD, H), _f32)],
        )(h, batch3, ug_all[l], W1a, W1b)

        ga, gb = _sc_gather2(ha2, hb, row_g, col_g)
        ga = ga.reshape(E_PAD // 2, 2 * H)
        gb = gb.reshape(E_PAD // 2, 2 * H)

        espec = pl.BlockSpec((TILE_E // 2, 2 * H), lambda i: (i, 0))
        if l == 0:
            ea = pl.pallas_call(
                _edge0_body,
                grid=(E_BLKS,),
                in_specs=[espec, espec,
                          pl.BlockSpec((TILE_E // 2, 32), lambda i: (i, 0)),
                          _full((32, 2 * H)), _full((1, 2 * H)),
                          _full((2 * H, 2 * H)), _full((1, 2 * H)),
                          _full((2 * H, 2 * H)), _full((2 * H, 2 * H)),
                          _full((1, 2 * H))],
                out_specs=espec,
                out_shape=jax.ShapeDtypeStruct((E_PAD // 2, 2 * H), _f32),
            )(ga, gb, attr2, bd2(ee_W1), b2x(ee_b1), bd2(ee_W2), b2x(ee_b2),
              bd2(W1c), bd2(pe_W2[l]), b2x(pe_b2[l]))
        else:
            ea = pl.pallas_call(
                _edge_body,
                grid=(E_BLKS,),
                in_specs=[espec, espec, espec,
                          _full((2 * H, 2 * H)), _full((2 * H, 2 * H)),
                          _full((1, 2 * H))],
                out_specs=espec,
                out_shape=jax.ShapeDtypeStruct((E_PAD // 2, 2 * H), _f32),
            )(ga, gb, ea, bd2(W1c), bd2(pe_W2[l]), b2x(pe_b2[l]))

        aggp = _sc_scatter_add(ea.reshape(E_PAD, H), row_s,
                               zeros_nh).reshape(2, N_PAD, H)

        Wn = pn_W1[l]
        h = pl.pallas_call(
            _node_body,
            grid=(N_BLKS,),
            in_specs=[_row_spec(),
                      pl.BlockSpec((2, TILE_N, H), lambda i: (0, i, 0)),
                      _b3_spec(), _full((B, H)),
                      _full((H, H)), _full((H, H)),
                      _full((H, H)), _full((1, H))],
            out_specs=_row_spec(),
            out_shape=jax.ShapeDtypeStruct((N_PAD, H), _f32),
        )(h, aggp, batch3, attc_all[l], Wn[:H], Wn[H:2 * H],
          pn_W2[l], r2(pn_b2[l]))

    out = pl.pallas_call(
        _encode_body,
        grid=(N_BLKS,),
        in_specs=[_row_spec(), _full((H, H)), _full((1, H)),
                  _full((H, D_OUT)), _full((1, D_OUT))],
        out_specs=_row_spec(D_OUT),
        out_shape=jax.ShapeDtypeStruct((N_PAD, D_OUT), _f32),
    )(h, dec_W1, r2(dec_b1), dec_W2, r2(dec_b2))

    return out[:N]
